# baseline (speedup 1.0000x reference)
import functools

import jax
import jax.numpy as jnp
from jax import lax
from jax.experimental import pallas as pl
from jax.experimental.pallas import tpu as pltpu
from jax.experimental.pallas import tpu_sc as plsc

_NC = 2
_NS = 16
_NW = _NC * _NS
_L = 16


@functools.lru_cache(maxsize=None)
def _build(B, C, D, V, interpret=False):
    BPW = B // _NW
    SUB = min(128, BPW)
    NSUB = BPW // SUB
    NBLK = SUB // _L
    KD = D // _L
    CR = SUB * C
    IR = CR // 128

    mesh = plsc.VectorSubcoreMesh(
        core_axis_name="c", subcore_axis_name="s",
        num_cores=_NC, num_subcores=_NS)

    @functools.partial(
        pl.kernel,
        out_type=jax.ShapeDtypeStruct((_NW * NSUB, C, SUB), jnp.float32),
        mesh=mesh,
        interpret=interpret,
        compiler_params=pltpu.CompilerParams(
            needs_layout_passes=False, use_tc_tiling_on_sc=False),
        scratch_types=[
            pltpu.VMEM((NSUB, SUB), jnp.int32),
            pltpu.VMEM((NSUB * IR, 128), jnp.int32),
            pltpu.VMEM((SUB, D), jnp.float32),
            pltpu.VMEM((CR, D), jnp.float32),
            pltpu.VMEM((C, SUB), jnp.float32),
            pltpu.SemaphoreType.DMA,
            pltpu.SemaphoreType.DMA,
        ],
    )
    def sckern(t_hbm, ctx_hbm, tt_hbm, ct_hbm, out_hbm,
               tidx, cidx, trows, crows, dots, sem_i, sem_g):
        cid = lax.axis_index("c")
        sid = lax.axis_index("s")
        wid = sid * _NC + cid
        base = wid * BPW

        idescs = []
        for j in range(NSUB):
            idescs.append(pltpu.async_copy(
                t_hbm.at[pl.ds(base + j * SUB, SUB)], tidx.at[j], sem_i))
        for r in range(NSUB * IR):
            idescs.append(pltpu.async_copy(
                ctx_hbm.at[pl.ds(base * C + r * 128, 128)], cidx.at[r], sem_i))
        for d in idescs:
            d.wait()

        iota = lax.iota(jnp.int32, _L)

        @pl.loop(0, NSUB)
        def _sub(j):
            gdescs = [pltpu.async_copy(tt_hbm.at[tidx.at[j]], trows, sem_g)]
            for k in range(IR):
                gdescs.append(pltpu.async_copy(
                    ct_hbm.at[cidx.at[j * IR + k]],
                    crows.at[pl.ds(k * 128, 128)], sem_g))
            for d in gdescs:
                d.wait()

            @pl.loop(0, NBLK)
            def _blk(blk):
                acc = [jnp.zeros((_L,), jnp.float32) for _ in range(C)]
                for i in range(_L):
                    lb = blk * _L + i
                    w = [trows[lb, pl.ds(k * _L, _L)] for k in range(KD)]
                    for c in range(C):
                        row = lb * C + c
                        p = w[0] * crows[row, pl.ds(0, _L)]
                        for k in range(1, KD):
                            p = p + w[k] * crows[row, pl.ds(k * _L, _L)]
                        acc[c] = jnp.where(iota == i, jnp.sum(p), acc[c])
                for c in range(C):
                    dots[c, pl.ds(blk * _L, _L)] = acc[c]

            pltpu.sync_copy(dots, out_hbm.at[wid * NSUB + j])

    return sckern


def kernel(target, context, target_table, context_table):
    B, C = context.shape
    V, D = target_table.shape
    sck = _build(B, C, D, V)
    out = sck(target.astype(jnp.int32), context.reshape(-1).astype(jnp.int32),
              target_table, context_table)
    sub = out.shape[-1]
    return (out.reshape(B // sub, C, sub)
               .transpose(0, 2, 1)
               .reshape(B, C))
